# Initial kernel scaffold; baseline (speedup 1.0000x reference)
import functools

import jax
import jax.numpy as jnp
from jax import lax
from jax.experimental import pallas as pl
from jax.experimental.pallas import tpu as pltpu
from jax.experimental.pallas import tpu_sc as plsc

_N = 10000
_D = 128
_E = 320000
_CHUNK = 128
_ROWS = _E // _CHUNK
_NW = 32
_FULL = _ROWS // _NW
_EXTRA = _ROWS - _FULL * _NW
_NPS = _N // 16


def _matmul_bias(x, W, b):
    def body(x_ref, w_ref, b_ref, o_ref):
        o_ref[...] = (
            jnp.dot(x_ref[...], w_ref[...], preferred_element_type=jnp.float32)
            + b_ref[...]
        )

    return pl.pallas_call(
        body, out_shape=jax.ShapeDtypeStruct((_N, _D), jnp.float32)
    )(x, W, b.reshape(1, _D))


def _combine_ln_relu_matmul(part, g, be, W, b):
    def body(p_ref, g_ref, be_ref, w_ref, b_ref, o_ref):
        a = p_ref[0] + p_ref[1]
        mu = jnp.mean(a, axis=1, keepdims=True)
        c = a - mu
        var = jnp.mean(c * c, axis=1, keepdims=True)
        h = jnp.maximum(c * lax.rsqrt(var + 1e-5) * g_ref[...] + be_ref[...], 0.0)
        o_ref[...] = (
            jnp.dot(h, w_ref[...], preferred_element_type=jnp.float32) + b_ref[...]
        )

    return pl.pallas_call(
        body, out_shape=jax.ShapeDtypeStruct((_N, _D), jnp.float32)
    )(part, g.reshape(1, _D), be.reshape(1, _D), W, b.reshape(1, _D))


def _combine_ln_relu(part, g, be):
    def body(p_ref, g_ref, be_ref, o_ref):
        a = p_ref[0] + p_ref[1]
        mu = jnp.mean(a, axis=1, keepdims=True)
        c = a - mu
        var = jnp.mean(c * c, axis=1, keepdims=True)
        o_ref[...] = jnp.maximum(
            c * lax.rsqrt(var + 1e-5) * g_ref[...] + be_ref[...], 0.0
        )

    return pl.pallas_call(
        body, out_shape=jax.ShapeDtypeStruct((_N, _D), jnp.float32)
    )(part, g.reshape(1, _D), be.reshape(1, _D))


def _edge_aggregate(hl, src_r, dst_r, ew_r, zrows):
    mesh = plsc.VectorSubcoreMesh(core_axis_name="c", subcore_axis_name="s")

    @functools.partial(
        pl.kernel,
        out_type=jax.ShapeDtypeStruct((2, _N, _D), jnp.float32),
        mesh=mesh,
        scratch_types=[
            pltpu.VMEM((_FULL + 1, _CHUNK), jnp.int32),
            pltpu.VMEM((_FULL + 1, _CHUNK), jnp.int32),
            pltpu.VMEM((_FULL + 1, _CHUNK), jnp.float32),
            pltpu.VMEM((_CHUNK, _D), jnp.float32),
            pltpu.VMEM_SHARED((_N, _D), jnp.float32),
            pltpu.SemaphoreType.DMA,
        ],
    )
    def k(hl_h, src_h, dst_h, ew_h, z_h, out_h, src_v, dst_v, ew_v, rows_v,
          agg_sh, sem):
        cid = lax.axis_index("c")
        sid = lax.axis_index("s")
        w = sid * 2 + cid
        base = w * _FULL

        pltpu.sync_copy(z_h, agg_sh.at[pl.ds(sid * _NPS, _NPS)])

        pltpu.sync_copy(src_h.at[pl.ds(base, _FULL)], src_v.at[pl.ds(0, _FULL)])
        pltpu.sync_copy(dst_h.at[pl.ds(base, _FULL)], dst_v.at[pl.ds(0, _FULL)])
        pltpu.sync_copy(ew_h.at[pl.ds(base, _FULL)], ew_v.at[pl.ds(0, _FULL)])
        has_extra = w < _EXTRA

        @pl.when(has_extra)
        def _():
            eb = _FULL * _NW + w
            pltpu.sync_copy(src_h.at[pl.ds(eb, 1)], src_v.at[pl.ds(_FULL, 1)])
            pltpu.sync_copy(dst_h.at[pl.ds(eb, 1)], dst_v.at[pl.ds(_FULL, 1)])
            pltpu.sync_copy(ew_h.at[pl.ds(eb, 1)], ew_v.at[pl.ds(_FULL, 1)])

        plsc.subcore_barrier()

        def process(j):
            pltpu.async_copy(hl_h.at[src_v.at[j]], rows_v, sem).wait()

            def scale(e, carry):
                jv = jnp.broadcast_to(j, (16,)).astype(jnp.int32)
                ev = jnp.broadcast_to(e, (16,)).astype(jnp.int32)
                wv = plsc.load_gather(ew_v, [jv, ev])
                for q in range(_D // 16):
                    sl = pl.ds(q * 16, 16)
                    rows_v[e, sl] = rows_v[e, sl] * wv
                return carry

            lax.fori_loop(0, _CHUNK, scale, 0)

            pltpu.sync_copy(rows_v, agg_sh.at[dst_v.at[j]], add=True)

        lax.fori_loop(0, _FULL, lambda j, c: (process(j), c)[1], 0)

        @pl.when(has_extra)
        def _():
            process(jnp.int32(_FULL))

        plsc.subcore_barrier()

        pltpu.sync_copy(
            agg_sh.at[pl.ds(sid * _NPS, _NPS)],
            out_h.at[cid, pl.ds(sid * _NPS, _NPS)],
        )

    return k(hl, src_r, dst_r, ew_r, zrows)


def kernel(x, edge_index, edge_weight, W1, b1, W2, b2, g1, be1, g2, be2):
    src_r = edge_index[0].reshape(_ROWS, _CHUNK)
    dst_r = edge_index[1].reshape(_ROWS, _CHUNK)
    ew_r = edge_weight.reshape(_ROWS, _CHUNK)
    zrows = jnp.zeros((_NPS, _D), jnp.float32)

    hl1 = _matmul_bias(x, W1, b1)
    part1 = _edge_aggregate(hl1, src_r, dst_r, ew_r, zrows)
    hl2 = _combine_ln_relu_matmul(part1, g1, be1, W2, b2)
    part2 = _edge_aggregate(hl2, src_r, dst_r, ew_r, zrows)
    return _combine_ln_relu(part2, g2, be2)


# speedup vs baseline: 3.6132x; 3.6132x over previous
import functools

import jax
import jax.numpy as jnp
from jax import lax
from jax.experimental import pallas as pl
from jax.experimental.pallas import tpu as pltpu
from jax.experimental.pallas import tpu_sc as plsc

_N = 10000
_D = 128
_E = 320000
_CHUNK = 128
_NW = 32
_RP = 2560
_PW = _RP // _NW
_NP = 10112
_NPS = _NP // 16
_PPH = 40


def _matmul_bias(x, W, b):
    def body(x_ref, w_ref, b_ref, o_ref):
        o_ref[...] = (
            jnp.dot(x_ref[...], w_ref[...], preferred_element_type=jnp.float32)
            + b_ref[...]
        )

    return pl.pallas_call(
        body, out_shape=jax.ShapeDtypeStruct((_N, _D), jnp.float32)
    )(x, W, b.reshape(1, _D))


def _combine_ln_relu_matmul(part, g, be, W, b):
    def body(p_ref, g_ref, be_ref, w_ref, b_ref, o_ref):
        a = p_ref[0, : _N] + p_ref[1, : _N]
        mu = jnp.mean(a, axis=1, keepdims=True)
        c = a - mu
        var = jnp.mean(c * c, axis=1, keepdims=True)
        h = jnp.maximum(c * lax.rsqrt(var + 1e-5) * g_ref[...] + be_ref[...], 0.0)
        o_ref[...] = (
            jnp.dot(h, w_ref[...], preferred_element_type=jnp.float32) + b_ref[...]
        )

    return pl.pallas_call(
        body, out_shape=jax.ShapeDtypeStruct((_N, _D), jnp.float32)
    )(part, g.reshape(1, _D), be.reshape(1, _D), W, b.reshape(1, _D))


def _combine_ln_relu(part, g, be):
    def body(p_ref, g_ref, be_ref, o_ref):
        a = p_ref[0, : _N] + p_ref[1, : _N]
        mu = jnp.mean(a, axis=1, keepdims=True)
        c = a - mu
        var = jnp.mean(c * c, axis=1, keepdims=True)
        o_ref[...] = jnp.maximum(
            c * lax.rsqrt(var + 1e-5) * g_ref[...] + be_ref[...], 0.0
        )

    return pl.pallas_call(
        body, out_shape=jax.ShapeDtypeStruct((_N, _D), jnp.float32)
    )(part, g.reshape(1, _D), be.reshape(1, _D))


def _edge_aggregate(hl, src_r, dst_r, ew_r, zrows):
    mesh = plsc.VectorSubcoreMesh(core_axis_name="c", subcore_axis_name="s")

    @functools.partial(
        pl.kernel,
        out_type=jax.ShapeDtypeStruct((2, _NP, _D), jnp.float32),
        mesh=mesh,
        scratch_types=[
            pltpu.VMEM((_PPH, _CHUNK), jnp.int32),
            pltpu.VMEM((_PPH, _CHUNK), jnp.int32),
            pltpu.VMEM((_PPH, _CHUNK), jnp.float32),
            pltpu.VMEM((_CHUNK, _D), jnp.float32),
            pltpu.VMEM((_CHUNK, _D), jnp.float32),
            pltpu.VMEM_SHARED((_NP, _D), jnp.float32),
            pltpu.SemaphoreType.DMA,
            pltpu.SemaphoreType.DMA,
            pltpu.SemaphoreType.DMA,
            pltpu.SemaphoreType.DMA,
        ],
    )
    def k(hl_h, src_h, dst_h, ew_h, z_h, out_h, src_v, dst_v, ew_v, rows_a,
          rows_b, agg_sh, sga, sgb, ssa, ssb):
        cid = lax.axis_index("c")
        sid = lax.axis_index("s")
        w = sid * 2 + cid
        base = w * _PW

        pltpu.sync_copy(z_h, agg_sh.at[pl.ds(sid * _NPS, _NPS)])

        plsc.subcore_barrier()

        def start_gather(j, rows_v, sg):
            pltpu.async_copy(hl_h.at[src_v.at[j]], rows_v, sg)

        def scale(j, rows_v):
            def scale_group(g, c2):
                wgrp = ew_v[j, pl.ds(g * 16, 16)]
                for l in range(16):
                    wv = jnp.broadcast_to(wgrp[l], (16,))
                    e = g * 16 + l
                    for q in range(_D // 16):
                        sl = pl.ds(q * 16, 16)
                        rows_v[e, sl] = rows_v[e, sl] * wv
                return c2

            lax.fori_loop(0, _CHUNK // 16, scale_group, 0)

        def half(j, rows_v, sg, ss):
            pltpu.make_async_copy(hl_h.at[src_v.at[j]], rows_v, sg).wait()
            scale(j, rows_v)
            pltpu.async_copy(rows_v, agg_sh.at[dst_v.at[j]], ss, add=True)

        for p in range(_PW // _PPH):
            pltpu.sync_copy(src_h.at[pl.ds(base + p * _PPH, _PPH)], src_v)
            pltpu.sync_copy(dst_h.at[pl.ds(base + p * _PPH, _PPH)], dst_v)
            pltpu.sync_copy(ew_h.at[pl.ds(base + p * _PPH, _PPH)], ew_v)
            start_gather(0, rows_a, sga)
            start_gather(1, rows_b, sgb)

            def body(g, carry):
                jj = g * 2
                half(jj, rows_a, sga, ssa)
                half(jj + 1, rows_b, sgb, ssb)
                pltpu.make_async_copy(
                    rows_a, agg_sh.at[dst_v.at[jj]], ssa).wait()

                @pl.when(jj + 2 < _PPH)
                def _():
                    start_gather(jj + 2, rows_a, sga)

                pltpu.make_async_copy(
                    rows_b, agg_sh.at[dst_v.at[jj + 1]], ssb).wait()

                @pl.when(jj + 3 < _PPH)
                def _():
                    start_gather(jj + 3, rows_b, sgb)

                return carry

            lax.fori_loop(0, _PPH // 2, body, 0)

        plsc.subcore_barrier()

        pltpu.sync_copy(
            agg_sh.at[pl.ds(sid * _NPS, _NPS)],
            out_h.at[cid, pl.ds(sid * _NPS, _NPS)],
        )

    return k(hl, src_r, dst_r, ew_r, zrows)


def _pad_edges(edge_index, edge_weight):
    pad = _RP * _CHUNK - _E
    src = jnp.concatenate([edge_index[0], jnp.zeros((pad,), jnp.int32)])
    dst = jnp.concatenate([edge_index[1], jnp.zeros((pad,), jnp.int32)])
    ew = jnp.concatenate([edge_weight, jnp.zeros((pad,), jnp.float32)])
    return (src.reshape(_RP, _CHUNK), dst.reshape(_RP, _CHUNK),
            ew.reshape(_RP, _CHUNK))


def kernel(x, edge_index, edge_weight, W1, b1, W2, b2, g1, be1, g2, be2):
    src_r, dst_r, ew_r = _pad_edges(edge_index, edge_weight)
    zrows = jnp.zeros((_NPS, _D), jnp.float32)

    hl1 = _matmul_bias(x, W1, b1)
    part1 = _edge_aggregate(hl1, src_r, dst_r, ew_r, zrows)
    hl2 = _combine_ln_relu_matmul(part1, g1, be1, W2, b2)
    part2 = _edge_aggregate(hl2, src_r, dst_r, ew_r, zrows)
    return _combine_ln_relu(part2, g2, be2)
